# baseline (speedup 1.0000x reference)
import functools

import jax
import jax.numpy as jnp
from jax.experimental import pallas as pl
from jax.experimental.pallas import tpu as pltpu


def _sigmoid(z):
    return 1.0 / (1.0 + jnp.exp(-z))


def _cbam_kernel(nb, w, k, inv_s, inv_c,
                 wsp_ref, x_ref, wdt_ref, wut_ref,
                 o_ref,
                 pbuf_ref):
    f32 = jnp.float32
    s = x_ref.shape[2]
    x = x_ref[...].astype(f32)

    mx = jnp.max(x, axis=2)
    av = jnp.sum(x, axis=2) * inv_s
    pooled = jnp.concatenate([mx, av], axis=0)
    hid = jnp.maximum(
        jnp.dot(pooled, wdt_ref[...], preferred_element_type=f32), 0.0)
    z = jnp.dot(hid, wut_ref[...], preferred_element_type=f32)
    gate = _sigmoid(z[:nb] + z[nb:])

    x1 = x * gate[:, :, None]

    cmax = jnp.max(x1, axis=1)
    cmean = jnp.sum(x1, axis=1) * inv_c

    pbuf_ref[...] = jnp.zeros_like(pbuf_ref)
    pbuf_ref[0, :, 128:128 + s] = cmax
    pbuf_ref[1, :, 128:128 + s] = cmean

    col = jax.lax.broadcasted_iota(jnp.int32, (1, s), 1) % w
    half = k // 2
    acc = jnp.zeros((nb, s), f32)
    for dx in range(k):
        t = jnp.zeros((nb, s), f32)
        for ch in range(2):
            for dy in range(k):
                off = 128 + (dy - half) * w + (dx - half)
                t = t + wsp_ref[ch * k * k + dy * k + dx] * pbuf_ref[ch, :, off:off + s]
        m = (col + (dx - half) >= 0) & (col + (dx - half) < w)
        acc = acc + jnp.where(m, t, 0.0)
    sgate = _sigmoid(acc)

    o_ref[...] = (x1 * sgate[:, None, :]).astype(o_ref.dtype)


def kernel(x_nchw, w_mlp_down, w_mlp_up, w_spatial):
    N, C, H, W = x_nchw.shape
    Cr = w_mlp_down.shape[0]
    K = w_spatial.shape[-1]
    S = H * W
    NB = 8
    f32 = jnp.float32

    x_ncs = x_nchw.reshape(N, C, S)
    wdt = w_mlp_down.reshape(Cr, C).T.astype(f32)
    wut = w_mlp_up.reshape(C, Cr).T.astype(f32)
    wsp = w_spatial.reshape(-1).astype(f32)

    kern = functools.partial(_cbam_kernel, NB, W, K, 1.0 / S, 1.0 / C)
    out_ncs = pl.pallas_call(
        kern,
        out_shape=jax.ShapeDtypeStruct((N, C, S), x_nchw.dtype),
        grid=(N // NB,),
        in_specs=[pl.BlockSpec(memory_space=pltpu.MemorySpace.SMEM),
                  pl.BlockSpec((NB, C, S), lambda n: (n, 0, 0)),
                  pl.BlockSpec((C, Cr), lambda n: (0, 0)),
                  pl.BlockSpec((Cr, C), lambda n: (0, 0))],
        out_specs=pl.BlockSpec((NB, C, S), lambda n: (n, 0, 0)),
        scratch_shapes=[pltpu.VMEM((2, NB, S + 256), f32)],
        compiler_params=pltpu.CompilerParams(
            dimension_semantics=("arbitrary",),
            vmem_limit_bytes=96 * 1024 * 1024),
    )(wsp, x_ncs, wdt, wut)

    return out_ncs.reshape(N, C, H, W)


# speedup vs baseline: 1.0052x; 1.0052x over previous
import functools

import jax
import jax.numpy as jnp
from jax.experimental import pallas as pl
from jax.experimental.pallas import tpu as pltpu


def _sigmoid(z):
    return 1.0 / (1.0 + jnp.exp(-z))


def _cbam_kernel(nb, w, k, inv_s, inv_c,
                 wsp_ref, x_ref, wdt_ref, wut_ref,
                 o_ref,
                 pbuf_ref):
    f32 = jnp.float32
    s = x_ref.shape[2]
    x = x_ref[...].astype(f32)

    mx = jnp.max(x, axis=2)
    av = jnp.sum(x, axis=2) * inv_s
    pooled = jnp.concatenate([mx, av], axis=0)
    hid = jnp.maximum(
        jnp.dot(pooled, wdt_ref[...], preferred_element_type=f32), 0.0)
    z = jnp.dot(hid, wut_ref[...], preferred_element_type=f32)
    gate = _sigmoid(z[:nb] + z[nb:])

    x1 = x * gate[:, :, None]

    cmax = jnp.max(x1, axis=1)
    cmean = jnp.sum(x1, axis=1) * inv_c

    pbuf_ref[...] = jnp.zeros_like(pbuf_ref)
    pbuf_ref[0, :, 128:128 + s] = cmax
    pbuf_ref[1, :, 128:128 + s] = cmean

    col = jax.lax.broadcasted_iota(jnp.int32, (1, s), 1) % w
    half = k // 2
    acc = jnp.zeros((nb, s), f32)
    for dx in range(k):
        t = jnp.zeros((nb, s), f32)
        for ch in range(2):
            for dy in range(k):
                off = 128 + (dy - half) * w + (dx - half)
                t = t + wsp_ref[ch * k * k + dy * k + dx] * pbuf_ref[ch, :, off:off + s]
        m = (col + (dx - half) >= 0) & (col + (dx - half) < w)
        acc = acc + jnp.where(m, t, 0.0)
    sgate = _sigmoid(acc)

    o_ref[...] = (x1 * sgate[:, None, :]).astype(o_ref.dtype)


def kernel(x_nchw, w_mlp_down, w_mlp_up, w_spatial):
    N, C, H, W = x_nchw.shape
    Cr = w_mlp_down.shape[0]
    K = w_spatial.shape[-1]
    S = H * W
    NB = 8
    f32 = jnp.float32

    x_ncs = x_nchw.reshape(N, C, S)
    wdt = w_mlp_down.reshape(Cr, C).T.astype(f32)
    wut = w_mlp_up.reshape(C, Cr).T.astype(f32)
    wsp = w_spatial.reshape(-1).astype(f32)

    kern = functools.partial(_cbam_kernel, NB, W, K, 1.0 / S, 1.0 / C)
    out_ncs = pl.pallas_call(
        kern,
        out_shape=jax.ShapeDtypeStruct((N, C, S), x_nchw.dtype),
        grid=(N // NB,),
        in_specs=[pl.BlockSpec(memory_space=pltpu.MemorySpace.SMEM),
                  pl.BlockSpec((NB, C, S), lambda n: (n, 0, 0)),
                  pl.BlockSpec((C, Cr), lambda n: (0, 0)),
                  pl.BlockSpec((Cr, C), lambda n: (0, 0))],
        out_specs=pl.BlockSpec((NB, C, S), lambda n: (n, 0, 0)),
        scratch_shapes=[pltpu.VMEM((2, NB, S + 256), f32)],
        compiler_params=pltpu.CompilerParams(
            dimension_semantics=("parallel",),
            vmem_limit_bytes=96 * 1024 * 1024),
    )(wsp, x_ncs, wdt, wut)

    return out_ncs.reshape(N, C, H, W)
